# Initial kernel scaffold; baseline (speedup 1.0000x reference)
import jax
import jax.numpy as jnp
from jax.experimental import pallas as pl


def kernel(user_ids, item_ids, concept_ids, users_emb_weight, item_resp_weight, W, mask, nb_modalities):
    raise NotImplementedError("write your pallas kernel here")


# speedup vs baseline: 7.5724x; 7.5724x over previous
import functools

import jax
import jax.numpy as jnp
from jax import lax
from jax.experimental import pallas as pl
from jax.experimental.pallas import tpu as pltpu
from jax.experimental.pallas import tpu_sc as plsc


def _sc_gather(user_ids, item_ids, utable, itable2, nbtable):
    b = user_ids.shape[0]
    cn = utable.shape[1]
    row = itable2.shape[1]
    info = plsc.get_sparse_core_info()
    nw = info.num_cores * info.num_subcores
    b_per_w = b // nw
    ch = 128
    nch = b_per_w // ch
    mesh = plsc.VectorSubcoreMesh(core_axis_name="c", subcore_axis_name="s")

    @functools.partial(
        pl.kernel,
        mesh=mesh,
        compiler_params=pltpu.CompilerParams(use_tc_tiling_on_sc=False),
        out_type=[
            jax.ShapeDtypeStruct((b, cn), jnp.float32),
            jax.ShapeDtypeStruct((b, row), jnp.float32),
            jax.ShapeDtypeStruct((b,), jnp.int32),
        ],
        scratch_types=[
            pltpu.VMEM((ch,), jnp.int32),
            pltpu.VMEM((ch, cn), jnp.float32),
            pltpu.VMEM((ch, row), jnp.float32),
            pltpu.VMEM((ch,), jnp.int32),
            pltpu.SemaphoreType.DMA,
            pltpu.SemaphoreType.DMA,
            pltpu.SemaphoreType.DMA,
        ],
    )
    def k(uid_hbm, iid_hbm, ut_hbm, it_hbm, nb_hbm, uout, iout, nbout,
          idx_v, urow_v, irow_v, nb_v, s1, s2, s3):
        wid = lax.axis_index("s") * info.num_cores + lax.axis_index("c")
        for c in range(nch):
            base = wid * b_per_w + c * ch
            pltpu.sync_copy(uid_hbm.at[pl.ds(base, ch)], idx_v)
            pltpu.async_copy(ut_hbm.at[idx_v], urow_v, s1).wait()
            pltpu.sync_copy(urow_v, uout.at[pl.ds(base, ch)])
            pltpu.sync_copy(iid_hbm.at[pl.ds(base, ch)], idx_v)
            a1 = pltpu.async_copy(it_hbm.at[idx_v], irow_v, s2)
            a2 = pltpu.async_copy(nb_hbm.at[idx_v], nb_v, s3)
            a1.wait()
            a2.wait()
            pltpu.sync_copy(irow_v, iout.at[pl.ds(base, ch)])
            pltpu.sync_copy(nb_v, nbout.at[pl.ds(base, ch)])

    return k(user_ids, item_ids, utable, itable2, nbtable)


def _tc_body(m, d_in, cn, bb, cid_ref, nbm_ref, u_ref, ir_ref, wf_ref, out_ref):
    cid = cid_ref[...]
    nbm = nbm_ref[...]
    u = u_ref[...]
    ir = ir_ref[...]
    wf = wf_ref[...]

    onehot = (lax.broadcasted_iota(jnp.int32, (bb, cn), 1) == cid)
    wtf = jnp.dot(onehot.astype(jnp.float32), wf,
                  preferred_element_type=jnp.float32)
    irb = ir.astype(jnp.bfloat16).astype(jnp.float32)

    minv = jnp.full((bb, 1), jnp.inf, dtype=jnp.float32)
    argm = jnp.zeros((bb, 1), dtype=jnp.int32)
    for s in range(1, m - 1):
        im = jnp.zeros((bb, cn), jnp.float32)
        for d in range(d_in):
            im = im + irb[:, s * d_in + d:s * d_in + d + 1] \
                * wtf[:, d * cn:(d + 1) * cn]
        diff = u - im
        p = jnp.sum(diff * diff, axis=1, keepdims=True)
        pv = jnp.where(nbm >= s, p, jnp.inf)
        better = pv < minv
        minv = jnp.where(better, pv, minv)
        argm = jnp.where(better, s, argm)

    resp = (argm - 1).astype(jnp.float32) / (nbm - 1).astype(jnp.float32) + 1.0
    out_ref[...] = resp


def kernel(user_ids, item_ids, concept_ids, users_emb_weight,
           item_resp_weight, W, mask, nb_modalities):
    b = user_ids.shape[0]
    m = mask.shape[1]
    item_n = nb_modalities.shape[0]
    cn = users_emb_weight.shape[1]
    d_in = item_resp_weight.shape[1]

    itable2 = item_resp_weight.reshape(item_n, m * d_in)
    u_g, ir_g, nb_g = _sc_gather(user_ids, item_ids, users_emb_weight,
                                 itable2, nb_modalities)

    bb = 256
    nblk = b // bb
    wf = W.reshape(cn, d_in * cn)
    cid2 = concept_ids.reshape(b, 1)
    nbm2 = nb_g.reshape(b, 1)

    out = pl.pallas_call(
        functools.partial(_tc_body, m, d_in, cn, bb),
        grid=(nblk,),
        in_specs=[
            pl.BlockSpec((bb, 1), lambda i: (i, 0)),
            pl.BlockSpec((bb, 1), lambda i: (i, 0)),
            pl.BlockSpec((bb, cn), lambda i: (i, 0)),
            pl.BlockSpec((bb, m * d_in), lambda i: (i, 0)),
            pl.BlockSpec((cn, d_in * cn), lambda i: (0, 0)),
        ],
        out_specs=pl.BlockSpec((bb, 1), lambda i: (i, 0)),
        out_shape=jax.ShapeDtypeStruct((b, 1), jnp.float32),
    )(cid2, nbm2, u_g, ir_g, wf)
    return out.reshape(b)


# speedup vs baseline: 7.5729x; 1.0001x over previous
import functools

import jax
import jax.numpy as jnp
from jax import lax
from jax.experimental import pallas as pl
from jax.experimental.pallas import tpu as pltpu
from jax.experimental.pallas import tpu_sc as plsc


def _sc_gather(user_ids, item_ids, idx14, utable, itable, nbtable):
    b = user_ids.shape[0]
    cn = utable.shape[1]
    d_in = itable.shape[1]
    mm = idx14.shape[0] // b
    info = plsc.get_sparse_core_info()
    nw = info.num_cores * info.num_subcores
    b_per_w = b // nw
    ch = 128
    nch = b_per_w // ch
    chi = ch * mm
    mesh = plsc.VectorSubcoreMesh(core_axis_name="c", subcore_axis_name="s")

    nrow = b * mm // ch

    @functools.partial(
        pl.kernel,
        mesh=mesh,
        compiler_params=pltpu.CompilerParams(use_tc_tiling_on_sc=False),
        out_type=[
            jax.ShapeDtypeStruct((b, cn), jnp.float32),
            jax.ShapeDtypeStruct((nrow, ch, d_in), jnp.float32),
            jax.ShapeDtypeStruct((b,), jnp.int32),
        ],
        scratch_types=[
            pltpu.VMEM((ch,), jnp.int32),
            pltpu.VMEM((ch,), jnp.int32),
            pltpu.VMEM((mm, ch), jnp.int32),
            pltpu.VMEM((ch, cn), jnp.float32),
            pltpu.VMEM((mm, ch, d_in), jnp.float32),
            pltpu.VMEM((ch,), jnp.int32),
            pltpu.SemaphoreType.DMA,
            pltpu.SemaphoreType.DMA,
            pltpu.SemaphoreType.DMA,
        ],
    )
    def k(uid_hbm, iid_hbm, idx14_hbm, ut_hbm, it_hbm, nb_hbm,
          uout, iout, nbout,
          uidx_v, iidx_v, idx14_v, urow_v, irow_v, nb_v, s1, s2, s3):
        wid = lax.axis_index("s") * info.num_cores + lax.axis_index("c")
        for c in range(nch):
            base = wid * b_per_w + c * ch
            r0 = base * mm // ch
            pltpu.sync_copy(uid_hbm.at[pl.ds(base, ch)], uidx_v)
            a0 = pltpu.async_copy(ut_hbm.at[uidx_v], urow_v, s1)
            pltpu.sync_copy(iid_hbm.at[pl.ds(base, ch)], iidx_v)
            a2 = pltpu.async_copy(nb_hbm.at[iidx_v], nb_v, s3)
            pltpu.sync_copy(idx14_hbm.at[pl.ds(r0, mm)], idx14_v)
            gathers = [
                pltpu.async_copy(it_hbm.at[idx14_v.at[k_]],
                                 irow_v.at[k_], s2)
                for k_ in range(mm)
            ]
            a0.wait()
            pltpu.sync_copy(urow_v, uout.at[pl.ds(base, ch)])
            a2.wait()
            pltpu.sync_copy(nb_v, nbout.at[pl.ds(base, ch)])
            for g in gathers:
                g.wait()
            pltpu.sync_copy(irow_v, iout.at[pl.ds(r0, mm)])

    idx14_2d = idx14.reshape(nrow, ch)
    return k(user_ids, item_ids, idx14_2d, utable, itable, nbtable)


def _tc_body(m, d_in, cn, bb, cid_ref, nbm_ref, u_ref, ir_ref, wf_ref, out_ref):
    cid = cid_ref[...]
    nbm = nbm_ref[...]
    u = u_ref[...]
    ir = ir_ref[...]
    wf = wf_ref[...]

    onehot = (lax.broadcasted_iota(jnp.int32, (bb, cn), 1) == cid)
    wtf = jnp.dot(onehot.astype(jnp.float32), wf,
                  preferred_element_type=jnp.float32)
    irb = ir.astype(jnp.bfloat16).astype(jnp.float32)

    minv = jnp.full((bb, 1), jnp.inf, dtype=jnp.float32)
    argm = jnp.zeros((bb, 1), dtype=jnp.int32)
    for s in range(1, m - 1):
        im = jnp.zeros((bb, cn), jnp.float32)
        for d in range(d_in):
            im = im + irb[:, s * d_in + d:s * d_in + d + 1] \
                * wtf[:, d * cn:(d + 1) * cn]
        diff = u - im
        p = jnp.sum(diff * diff, axis=1, keepdims=True)
        pv = jnp.where(nbm >= s, p, jnp.inf)
        better = pv < minv
        minv = jnp.where(better, pv, minv)
        argm = jnp.where(better, s, argm)

    resp = (argm - 1).astype(jnp.float32) / (nbm - 1).astype(jnp.float32) + 1.0
    out_ref[...] = resp


def kernel(user_ids, item_ids, concept_ids, users_emb_weight,
           item_resp_weight, W, mask, nb_modalities):
    b = user_ids.shape[0]
    m = mask.shape[1]
    item_n = nb_modalities.shape[0]
    cn = users_emb_weight.shape[1]
    d_in = item_resp_weight.shape[1]

    idx14 = (item_ids[:, None] * m + jnp.arange(m, dtype=item_ids.dtype)
             ).reshape(b * m)
    u_g, ir14_g, nb_g = _sc_gather(user_ids, item_ids, idx14,
                                   users_emb_weight, item_resp_weight,
                                   nb_modalities)
    ir_g = ir14_g.reshape(b, m * d_in)

    bb = 256
    nblk = b // bb
    wf = W.reshape(cn, d_in * cn)
    cid2 = concept_ids.reshape(b, 1)
    nbm2 = nb_g.reshape(b, 1)

    out = pl.pallas_call(
        functools.partial(_tc_body, m, d_in, cn, bb),
        grid=(nblk,),
        in_specs=[
            pl.BlockSpec((bb, 1), lambda i: (i, 0)),
            pl.BlockSpec((bb, 1), lambda i: (i, 0)),
            pl.BlockSpec((bb, cn), lambda i: (i, 0)),
            pl.BlockSpec((bb, m * d_in), lambda i: (i, 0)),
            pl.BlockSpec((cn, d_in * cn), lambda i: (0, 0)),
        ],
        out_specs=pl.BlockSpec((bb, 1), lambda i: (i, 0)),
        out_shape=jax.ShapeDtypeStruct((b, 1), jnp.float32),
    )(cid2, nbm2, u_g, ir_g, wf)
    return out.reshape(b)
